# Initial kernel scaffold; baseline (speedup 1.0000x reference)
import jax
import jax.numpy as jnp
from jax.experimental import pallas as pl


def kernel(boxes, scores):
    raise NotImplementedError("write your pallas kernel here")


# speedup vs baseline: 39.5691x; 39.5691x over previous
import functools

import jax
import jax.numpy as jnp
from jax import lax
from jax.experimental import pallas as pl
from jax.experimental.pallas import tpu as pltpu
from jax.experimental.pallas import tpu_sc as plsc

_N = 5000
_NP = 5120
_VEC = _NP // 16
_NS = 16
_VPW = _VEC // _NS
_PROB = 0.5
_IOU = 0.5


def _nms_body(x1h, y1h, x2h, y2h, sh, odh,
              ox1h, oy1h, ox2h, oy2h, osh,
              x1u, y1u, x2u, y2u, su, odv,
              x1s, y1s, x2s, y2s, ars, ss, fl, pubv, rda, rdb,
              sha, shb):
    sid = lax.axis_index("s")
    lanes = lax.broadcasted_iota(jnp.int32, (16,), 0)

    pltpu.sync_copy(x1h, x1u)
    pltpu.sync_copy(y1h, y1u)
    pltpu.sync_copy(x2h, x2u)
    pltpu.sync_copy(y2h, y2u)
    pltpu.sync_copy(sh, su)
    pltpu.sync_copy(odh, odv)

    def build(v, cnt):
        b = v * 16
        idx = odv[pl.ds(b, 16)]
        gx1 = plsc.load_gather(x1u, [idx])
        gy1 = plsc.load_gather(y1u, [idx])
        gx2 = plsc.load_gather(x2u, [idx])
        gy2 = plsc.load_gather(y2u, [idx])
        gs = plsc.load_gather(su, [idx])
        x1s[pl.ds(b, 16)] = gx1
        y1s[pl.ds(b, 16)] = gy1
        x2s[pl.ds(b, 16)] = gx2
        y2s[pl.ds(b, 16)] = gy2
        ss[pl.ds(b, 16)] = gs
        ars[pl.ds(b, 16)] = (gx2 - gx1) * (gy2 - gy1)
        valid = (gs >= _PROB).astype(jnp.int32)
        fl[pl.ds(b, 16)] = valid
        return cnt + jnp.sum(valid)

    nvalid = lax.fori_loop(0, _VEC, build, jnp.int32(0))
    nblk = (nvalid + 15) // 16

    pubv[...] = jnp.zeros((16,), jnp.int32)

    def zslot(t, c):
        pltpu.sync_copy(pubv, sha.at[sid + t * _NS])
        pltpu.sync_copy(pubv, shb.at[sid + t * _NS])
        return c

    lax.fori_loop(0, _VEC // _NS, zslot, 0)
    plsc.subcore_barrier()

    def ptag(k):
        return (k + 1) + (nvalid << 9)

    def block(k, carry):
        base = k * 16

        @pl.when(sid == lax.rem(k, _NS))
        def _():
            bx1 = x1s[pl.ds(base, 16)]
            by1 = y1s[pl.ds(base, 16)]
            bx2 = x2s[pl.ds(base, 16)]
            by2 = y2s[pl.ds(base, 16)]
            ba = ars[pl.ds(base, 16)]
            f0 = fl[pl.ds(base, 16)] > 0

            def cond(c):
                _, rem = c
                return jnp.any(rem)

            def step(c):
                f, rem = c
                iv = plsc.all_reduce_ffs(rem)
                gi = base + iv
                cx1 = plsc.load_gather(x1s, [gi])
                cy1 = plsc.load_gather(y1s, [gi])
                cx2 = plsc.load_gather(x2s, [gi])
                cy2 = plsc.load_gather(y2s, [gi])
                ca = plsc.load_gather(ars, [gi])
                ix1 = jnp.maximum(bx1, cx1)
                iy1 = jnp.maximum(by1, cy1)
                ix2 = jnp.minimum(bx2, cx2)
                iy2 = jnp.minimum(by2, cy2)
                inter = jnp.maximum(ix2 - ix1, 0.0) * jnp.maximum(iy2 - iy1, 0.0)
                union = ba + ca - inter
                iou = inter / jnp.maximum(union, 1e-9)
                supp = (iou > _IOU) & (lanes > iv)
                f = f & ~supp
                return f, f & (lanes > iv)

            ffin, _ = lax.while_loop(cond, step, (f0, f0))
            fi = ffin.astype(jnp.int32)
            fl[pl.ds(base, 16)] = fi
            pubv[...] = fi + (ptag(k) << 1)
            pltpu.sync_copy(pubv, sha.at[k])
            pltpu.sync_copy(pubv, shb.at[k])

        plsc.subcore_barrier()
        pltpu.sync_copy(sha.at[k], rda)
        pltpu.sync_copy(shb.at[k], rdb)
        va = rda[...]
        vb = rdb[...]
        oka = (va >> 1) == ptag(k)
        val = jnp.where(oka, va, vb)
        fl[pl.ds(base, 16)] = val & 1

        t0 = (k - sid + 16) // 16
        t1 = (nblk - sid + 15) // 16

        def tgt(t, c):
            tb = (sid + t * 16) * 16
            tx1 = x1s[pl.ds(tb, 16)]
            ty1 = y1s[pl.ds(tb, 16)]
            tx2 = x2s[pl.ds(tb, 16)]
            ty2 = y2s[pl.ds(tb, 16)]
            ta = ars[pl.ds(tb, 16)]
            acc = fl[pl.ds(tb, 16)] > 0
            for i in range(16):
                gi = jnp.full((16,), base + i, jnp.int32)
                cx1 = plsc.load_gather(x1s, [gi])
                cy1 = plsc.load_gather(y1s, [gi])
                cx2 = plsc.load_gather(x2s, [gi])
                cy2 = plsc.load_gather(y2s, [gi])
                ca = plsc.load_gather(ars, [gi])
                ki = plsc.load_gather(fl, [gi]) > 0
                ix1 = jnp.maximum(tx1, cx1)
                iy1 = jnp.maximum(ty1, cy1)
                ix2 = jnp.minimum(tx2, cx2)
                iy2 = jnp.minimum(ty2, cy2)
                inter = jnp.maximum(ix2 - ix1, 0.0) * jnp.maximum(iy2 - iy1, 0.0)
                union = ta + ca - inter
                iou = inter / jnp.maximum(union, 1e-9)
                acc = acc & ~((iou > _IOU) & ki)
            fl[pl.ds(tb, 16)] = acc.astype(jnp.int32)
            return c

        lax.fori_loop(t0, t1, tgt, 0)
        return carry

    lax.fori_loop(0, nblk, block, jnp.int32(0))

    def outw(v, c):
        b = (sid * _VPW + v) * 16
        f = fl[pl.ds(b, 16)] > 0
        ff = f.astype(jnp.float32)
        x1u[pl.ds(b, 16)] = x1s[pl.ds(b, 16)] * ff
        y1u[pl.ds(b, 16)] = y1s[pl.ds(b, 16)] * ff
        x2u[pl.ds(b, 16)] = x2s[pl.ds(b, 16)] * ff
        y2u[pl.ds(b, 16)] = y2s[pl.ds(b, 16)] * ff
        su[pl.ds(b, 16)] = jnp.where(f, ss[pl.ds(b, 16)], 0.0)
        return c

    lax.fori_loop(0, _VPW, outw, 0)
    rows = _VPW * 16
    o = sid * rows
    pltpu.sync_copy(x1u.at[pl.ds(o, rows)], ox1h.at[pl.ds(o, rows)])
    pltpu.sync_copy(y1u.at[pl.ds(o, rows)], oy1h.at[pl.ds(o, rows)])
    pltpu.sync_copy(x2u.at[pl.ds(o, rows)], ox2h.at[pl.ds(o, rows)])
    pltpu.sync_copy(y2u.at[pl.ds(o, rows)], oy2h.at[pl.ds(o, rows)])
    pltpu.sync_copy(su.at[pl.ds(o, rows)], osh.at[pl.ds(o, rows)])


@jax.jit
def kernel(boxes, scores):
    s = jnp.where(scores >= _PROB, scores, -jnp.inf)
    order = jnp.argsort(-s).astype(jnp.int32)
    pad = _NP - _N
    op = jnp.concatenate([order, jnp.arange(_N, _NP, dtype=jnp.int32)])
    sp = jnp.concatenate([s, jnp.full((pad,), -jnp.inf, jnp.float32)])
    bp = jnp.pad(boxes, ((0, pad), (0, 0)))
    x1, y1, x2, y2 = bp[:, 0], bp[:, 1], bp[:, 2], bp[:, 3]

    f32 = jax.ShapeDtypeStruct((_NP,), jnp.float32)
    call = pl.kernel(
        _nms_body,
        out_type=(f32, f32, f32, f32, f32),
        mesh=plsc.VectorSubcoreMesh(
            core_axis_name="c", subcore_axis_name="s",
            num_cores=1, num_subcores=_NS),
        scratch_types=[
            pltpu.VMEM((_NP,), jnp.float32),
            pltpu.VMEM((_NP,), jnp.float32),
            pltpu.VMEM((_NP,), jnp.float32),
            pltpu.VMEM((_NP,), jnp.float32),
            pltpu.VMEM((_NP,), jnp.float32),
            pltpu.VMEM((_NP,), jnp.int32),
            pltpu.VMEM((_NP,), jnp.float32),
            pltpu.VMEM((_NP,), jnp.float32),
            pltpu.VMEM((_NP,), jnp.float32),
            pltpu.VMEM((_NP,), jnp.float32),
            pltpu.VMEM((_NP,), jnp.float32),
            pltpu.VMEM((_NP,), jnp.float32),
            pltpu.VMEM((_NP,), jnp.int32),
            pltpu.VMEM((16,), jnp.int32),
            pltpu.VMEM((16,), jnp.int32),
            pltpu.VMEM((16,), jnp.int32),
            pltpu.VMEM_SHARED((_VEC, 16), jnp.int32),
            pltpu.VMEM_SHARED((_VEC, 16), jnp.int32),
        ],
        compiler_params=pltpu.CompilerParams(needs_layout_passes=False),
    )
    ox1, oy1, ox2, oy2, os_ = call(x1, y1, x2, y2, sp, op)
    kept_boxes = jnp.stack([ox1, oy1, ox2, oy2], axis=1)[:_N]
    kept_scores = os_[:_N]
    return kept_boxes, kept_scores


# speedup vs baseline: 42.6724x; 1.0784x over previous
import functools

import jax
import jax.numpy as jnp
from jax import lax
from jax.experimental import pallas as pl
from jax.experimental.pallas import tpu as pltpu
from jax.experimental.pallas import tpu_sc as plsc

_N = 5000
_NP = 5120
_VEC = _NP // 16
_NS = 16
_VPW = _VEC // _NS
_PROB = 0.5
_C24 = 2.0 ** 24
_C25 = 2.0 ** 25


def _take(v, idx):
    return v.at[idx].get(mode="promise_in_bounds")


def _supp_test(ix1, iy1, ix2, iy2, ua, ub):
    inter = jnp.maximum(ix2 - ix1, 0.0) * jnp.maximum(iy2 - iy1, 0.0)
    u = jnp.maximum(ua + ub - inter, 1e-9)
    big = inter * _C25
    a = u * _C24
    hi = a + u
    z = hi - a
    lo = u - z
    return (big > hi) | ((big == hi) & (lo < 0.0))


def _nms_body(x1h, y1h, x2h, y2h, sh, odh,
              ox1h, oy1h, ox2h, oy2h, osh,
              x1u, y1u, x2u, y2u, su, odv,
              x1s, y1s, x2s, y2s, ars, ss, fl, pubv, rda, rdb,
              sema, semb,
              sha, shb):
    sid = lax.axis_index("s")
    lanes = lax.broadcasted_iota(jnp.int32, (16,), 0)

    pltpu.sync_copy(x1h, x1u)
    pltpu.sync_copy(y1h, y1u)
    pltpu.sync_copy(x2h, x2u)
    pltpu.sync_copy(y2h, y2u)
    pltpu.sync_copy(sh, su)
    pltpu.sync_copy(odh, odv)

    def build(v, cnt):
        b = v * 16
        idx = odv[pl.ds(b, 16)]
        gx1 = plsc.load_gather(x1u, [idx])
        gy1 = plsc.load_gather(y1u, [idx])
        gx2 = plsc.load_gather(x2u, [idx])
        gy2 = plsc.load_gather(y2u, [idx])
        gs = plsc.load_gather(su, [idx])
        x1s[pl.ds(b, 16)] = gx1
        y1s[pl.ds(b, 16)] = gy1
        x2s[pl.ds(b, 16)] = gx2
        y2s[pl.ds(b, 16)] = gy2
        ss[pl.ds(b, 16)] = gs
        ars[pl.ds(b, 16)] = (gx2 - gx1) * (gy2 - gy1)
        valid = (gs >= _PROB).astype(jnp.int32)
        fl[pl.ds(b, 16)] = valid
        return cnt + jnp.sum(valid)

    nvalid = lax.fori_loop(0, _VEC, build, jnp.int32(0))
    nblk = (nvalid + 15) // 16

    pubv[...] = jnp.zeros((16,), jnp.int32)

    def zslot(t, c):
        pltpu.sync_copy(pubv, sha.at[sid + t * _NS])
        pltpu.sync_copy(pubv, shb.at[sid + t * _NS])
        return c

    lax.fori_loop(0, _VEC // _NS, zslot, 0)
    plsc.subcore_barrier()

    def ptag(k):
        return (k + 1) + (nvalid << 9)

    def block(k, carry):
        base = k * 16
        bx1 = x1s[pl.ds(base, 16)]
        by1 = y1s[pl.ds(base, 16)]
        bx2 = x2s[pl.ds(base, 16)]
        by2 = y2s[pl.ds(base, 16)]
        ba = ars[pl.ds(base, 16)]

        @pl.when(sid == lax.rem(k, _NS))
        def _():
            f0 = fl[pl.ds(base, 16)] > 0

            def cond(c):
                _, rem = c
                return jnp.any(rem)

            def step(c):
                f, rem = c
                iv = plsc.all_reduce_ffs(rem)
                supp = _supp_test(
                    jnp.maximum(bx1, _take(bx1, iv)),
                    jnp.maximum(by1, _take(by1, iv)),
                    jnp.minimum(bx2, _take(bx2, iv)),
                    jnp.minimum(by2, _take(by2, iv)),
                    ba, _take(ba, iv)) & (lanes > iv)
                f = f & ~supp
                return f, f & (lanes > iv)

            ffin, _ = lax.while_loop(cond, step, (f0, f0))
            fi = ffin.astype(jnp.int32)
            fl[pl.ds(base, 16)] = fi
            pubv[...] = fi + (ptag(k) << 1)
            da = pltpu.async_copy(pubv, sha.at[k], sema)
            db = pltpu.async_copy(pubv, shb.at[k], semb)
            da.wait()
            db.wait()

        plsc.subcore_barrier()
        da = pltpu.async_copy(sha.at[k], rda, sema)
        db = pltpu.async_copy(shb.at[k], rdb, semb)
        da.wait()
        db.wait()
        va = rda[...]
        vb = rdb[...]
        oka = (va >> 1) == ptag(k)
        val = jnp.where(oka, va, vb)
        fl[pl.ds(base, 16)] = val & 1
        kfv = val & 1

        t0 = (k - sid + 16) // 16
        t1 = (nblk - sid + 15) // 16

        def tgt(t, c):
            tb = (sid + t * 16) * 16
            tx1 = x1s[pl.ds(tb, 16)]
            ty1 = y1s[pl.ds(tb, 16)]
            tx2 = x2s[pl.ds(tb, 16)]
            ty2 = y2s[pl.ds(tb, 16)]
            ta = ars[pl.ds(tb, 16)]
            acc = fl[pl.ds(tb, 16)] > 0
            for i in range(16):
                ii = jnp.full((16,), i, jnp.int32)
                supp = _supp_test(
                    jnp.maximum(tx1, _take(bx1, ii)),
                    jnp.maximum(ty1, _take(by1, ii)),
                    jnp.minimum(tx2, _take(bx2, ii)),
                    jnp.minimum(ty2, _take(by2, ii)),
                    ta, _take(ba, ii)) & (_take(kfv, ii) > 0)
                acc = acc & ~supp
            fl[pl.ds(tb, 16)] = acc.astype(jnp.int32)
            return c

        lax.fori_loop(t0, t1, tgt, 0)
        return carry

    lax.fori_loop(0, nblk, block, jnp.int32(0))

    def outw(v, c):
        b = (sid * _VPW + v) * 16
        f = fl[pl.ds(b, 16)] > 0
        ff = f.astype(jnp.float32)
        x1u[pl.ds(b, 16)] = x1s[pl.ds(b, 16)] * ff
        y1u[pl.ds(b, 16)] = y1s[pl.ds(b, 16)] * ff
        x2u[pl.ds(b, 16)] = x2s[pl.ds(b, 16)] * ff
        y2u[pl.ds(b, 16)] = y2s[pl.ds(b, 16)] * ff
        su[pl.ds(b, 16)] = jnp.where(f, ss[pl.ds(b, 16)], 0.0)
        return c

    lax.fori_loop(0, _VPW, outw, 0)
    rows = _VPW * 16
    o = sid * rows
    pltpu.sync_copy(x1u.at[pl.ds(o, rows)], ox1h.at[pl.ds(o, rows)])
    pltpu.sync_copy(y1u.at[pl.ds(o, rows)], oy1h.at[pl.ds(o, rows)])
    pltpu.sync_copy(x2u.at[pl.ds(o, rows)], ox2h.at[pl.ds(o, rows)])
    pltpu.sync_copy(y2u.at[pl.ds(o, rows)], oy2h.at[pl.ds(o, rows)])
    pltpu.sync_copy(su.at[pl.ds(o, rows)], osh.at[pl.ds(o, rows)])


@jax.jit
def kernel(boxes, scores):
    s = jnp.where(scores >= _PROB, scores, -jnp.inf)
    order = jnp.argsort(-s).astype(jnp.int32)
    pad = _NP - _N
    op = jnp.concatenate([order, jnp.arange(_N, _NP, dtype=jnp.int32)])
    sp = jnp.concatenate([s, jnp.full((pad,), -jnp.inf, jnp.float32)])
    bp = jnp.pad(boxes, ((0, pad), (0, 0)))
    x1, y1, x2, y2 = bp[:, 0], bp[:, 1], bp[:, 2], bp[:, 3]

    f32 = jax.ShapeDtypeStruct((_NP,), jnp.float32)
    call = pl.kernel(
        _nms_body,
        out_type=(f32, f32, f32, f32, f32),
        mesh=plsc.VectorSubcoreMesh(
            core_axis_name="c", subcore_axis_name="s",
            num_cores=1, num_subcores=_NS),
        scratch_types=[
            pltpu.VMEM((_NP,), jnp.float32),
            pltpu.VMEM((_NP,), jnp.float32),
            pltpu.VMEM((_NP,), jnp.float32),
            pltpu.VMEM((_NP,), jnp.float32),
            pltpu.VMEM((_NP,), jnp.float32),
            pltpu.VMEM((_NP,), jnp.int32),
            pltpu.VMEM((_NP,), jnp.float32),
            pltpu.VMEM((_NP,), jnp.float32),
            pltpu.VMEM((_NP,), jnp.float32),
            pltpu.VMEM((_NP,), jnp.float32),
            pltpu.VMEM((_NP,), jnp.float32),
            pltpu.VMEM((_NP,), jnp.float32),
            pltpu.VMEM((_NP,), jnp.int32),
            pltpu.VMEM((16,), jnp.int32),
            pltpu.VMEM((16,), jnp.int32),
            pltpu.VMEM((16,), jnp.int32),
            pltpu.SemaphoreType.DMA,
            pltpu.SemaphoreType.DMA,
            pltpu.VMEM_SHARED((_VEC, 16), jnp.int32),
            pltpu.VMEM_SHARED((_VEC, 16), jnp.int32),
        ],
        compiler_params=pltpu.CompilerParams(needs_layout_passes=False),
    )
    ox1, oy1, ox2, oy2, os_ = call(x1, y1, x2, y2, sp, op)
    kept_boxes = jnp.stack([ox1, oy1, ox2, oy2], axis=1)[:_N]
    kept_scores = os_[:_N]
    return kept_boxes, kept_scores


# speedup vs baseline: 49.5525x; 1.1612x over previous
import functools

import jax
import jax.numpy as jnp
from jax import lax
from jax.experimental import pallas as pl
from jax.experimental.pallas import tpu as pltpu
from jax.experimental.pallas import tpu_sc as plsc

_N = 5000
_NP = 5120
_VEC = _NP // 16
_NS = 16
_VPW = _VEC // _NS
_PROB = 0.5
_C24 = 2.0 ** 24
_C25 = 2.0 ** 25


def _take(v, idx):
    return v.at[idx].get(mode="promise_in_bounds")


def _supp_test(ix1, iy1, ix2, iy2, ua, ub):
    inter = jnp.maximum(ix2 - ix1, 0.0) * jnp.maximum(iy2 - iy1, 0.0)
    u = jnp.maximum(ua + ub - inter, 1e-9)
    big = inter * _C25
    a = u * _C24
    hi = a + u
    z = hi - a
    lo = u - z
    return (big > hi) | ((big == hi) & (lo < 0.0))


def _nms_body(x1h, y1h, x2h, y2h, sh, odh,
              ox1h, oy1h, ox2h, oy2h, osh,
              x1u, y1u, x2u, y2u, su, odv,
              x1s, y1s, x2s, y2s, ars, ss, fl, pubv, rda, rdb,
              sema, semb,
              sha, shb):
    sid = lax.axis_index("s")
    lanes = lax.broadcasted_iota(jnp.int32, (16,), 0)

    pltpu.sync_copy(x1h, x1u)
    pltpu.sync_copy(y1h, y1u)
    pltpu.sync_copy(x2h, x2u)
    pltpu.sync_copy(y2h, y2u)
    pltpu.sync_copy(sh, su)
    pltpu.sync_copy(odh, odv)

    def build(v, cnt):
        b = v * 16
        idx = odv[pl.ds(b, 16)]
        gx1 = plsc.load_gather(x1u, [idx])
        gy1 = plsc.load_gather(y1u, [idx])
        gx2 = plsc.load_gather(x2u, [idx])
        gy2 = plsc.load_gather(y2u, [idx])
        gs = plsc.load_gather(su, [idx])
        x1s[pl.ds(b, 16)] = gx1
        y1s[pl.ds(b, 16)] = gy1
        x2s[pl.ds(b, 16)] = gx2
        y2s[pl.ds(b, 16)] = gy2
        ss[pl.ds(b, 16)] = gs
        ars[pl.ds(b, 16)] = (gx2 - gx1) * (gy2 - gy1)
        valid = (gs >= _PROB).astype(jnp.int32)
        fl[pl.ds(b, 16)] = valid
        return cnt + jnp.sum(valid)

    nvalid = lax.fori_loop(0, _VEC, build, jnp.int32(0))
    nrnd = (nvalid + 31) // 32

    pubv[...] = jnp.zeros((16,), jnp.int32)

    def zslot(t, c):
        pltpu.sync_copy(pubv, sha.at[sid + t * _NS])
        pltpu.sync_copy(pubv, shb.at[sid + t * _NS])
        return c

    lax.fori_loop(0, _VEC // _NS, zslot, 0)
    plsc.subcore_barrier()

    def ptag(r):
        return (r + 1) + (nvalid << 9)

    def _intra16(x1v, y1v, x2v, y2v, av, f0):
        def cond(c):
            _, rem = c
            return jnp.any(rem)

        def step(c):
            f, rem = c
            iv = plsc.all_reduce_ffs(rem)
            supp = _supp_test(
                jnp.maximum(x1v, _take(x1v, iv)),
                jnp.maximum(y1v, _take(y1v, iv)),
                jnp.minimum(x2v, _take(x2v, iv)),
                jnp.minimum(y2v, _take(y2v, iv)),
                av, _take(av, iv)) & (lanes > iv)
            f = f & ~supp
            return f, f & (lanes > iv)

        ffin, _ = lax.while_loop(cond, step, (f0, f0))
        return ffin

    def block(r, carry):
        b0 = r * 32
        b1 = b0 + 16
        ax1 = x1s[pl.ds(b0, 16)]
        ay1 = y1s[pl.ds(b0, 16)]
        ax2 = x2s[pl.ds(b0, 16)]
        ay2 = y2s[pl.ds(b0, 16)]
        aa = ars[pl.ds(b0, 16)]
        cx1 = x1s[pl.ds(b1, 16)]
        cy1 = y1s[pl.ds(b1, 16)]
        cx2 = x2s[pl.ds(b1, 16)]
        cy2 = y2s[pl.ds(b1, 16)]
        ca = ars[pl.ds(b1, 16)]

        @pl.when(sid == lax.rem(r, _NS))
        def _():
            fa = _intra16(ax1, ay1, ax2, ay2, aa, fl[pl.ds(b0, 16)] > 0)
            fai = fa.astype(jnp.int32)
            accc = fl[pl.ds(b1, 16)] > 0
            for i in range(16):
                ii = jnp.full((16,), i, jnp.int32)
                supp = _supp_test(
                    jnp.maximum(cx1, _take(ax1, ii)),
                    jnp.maximum(cy1, _take(ay1, ii)),
                    jnp.minimum(cx2, _take(ax2, ii)),
                    jnp.minimum(cy2, _take(ay2, ii)),
                    ca, _take(aa, ii)) & (_take(fai, ii) > 0)
                accc = accc & ~supp
            fc = _intra16(cx1, cy1, cx2, cy2, ca, accc)
            fci = fc.astype(jnp.int32)
            fl[pl.ds(b0, 16)] = fai
            fl[pl.ds(b1, 16)] = fci
            pubv[...] = fai + (fci << 1) + (ptag(r) << 2)
            da = pltpu.async_copy(pubv, sha.at[r], sema)
            db = pltpu.async_copy(pubv, shb.at[r], semb)
            da.wait()
            db.wait()

        plsc.subcore_barrier()
        da = pltpu.async_copy(sha.at[r], rda, sema)
        db = pltpu.async_copy(shb.at[r], rdb, semb)
        da.wait()
        db.wait()
        va = rda[...]
        vb = rdb[...]
        oka = (va >> 2) == ptag(r)
        val = jnp.where(oka, va, vb)
        kfa = val & 1
        kfc = (val >> 1) & 1
        fl[pl.ds(b0, 16)] = kfa
        fl[pl.ds(b1, 16)] = kfc

        t0 = (r - sid + 16) // 16
        t1 = (nrnd - sid + 15) // 16

        def tgt(t, c):
            tb = (sid + t * 16) * 32
            tb1 = tb + 16
            tx1 = x1s[pl.ds(tb, 16)]
            ty1 = y1s[pl.ds(tb, 16)]
            tx2 = x2s[pl.ds(tb, 16)]
            ty2 = y2s[pl.ds(tb, 16)]
            ta = ars[pl.ds(tb, 16)]
            ux1 = x1s[pl.ds(tb1, 16)]
            uy1 = y1s[pl.ds(tb1, 16)]
            ux2 = x2s[pl.ds(tb1, 16)]
            uy2 = y2s[pl.ds(tb1, 16)]
            ua = ars[pl.ds(tb1, 16)]
            acc0 = fl[pl.ds(tb, 16)] > 0
            acc1 = fl[pl.ds(tb1, 16)] > 0
            for i in range(32):
                ii = jnp.full((16,), i % 16, jnp.int32)
                if i < 16:
                    sx1, sy1, sx2, sy2, sa, sf = ax1, ay1, ax2, ay2, aa, kfa
                else:
                    sx1, sy1, sx2, sy2, sa, sf = cx1, cy1, cx2, cy2, ca, kfc
                px1 = _take(sx1, ii)
                py1 = _take(sy1, ii)
                px2 = _take(sx2, ii)
                py2 = _take(sy2, ii)
                pa = _take(sa, ii)
                pk = _take(sf, ii) > 0
                supp0 = _supp_test(
                    jnp.maximum(tx1, px1), jnp.maximum(ty1, py1),
                    jnp.minimum(tx2, px2), jnp.minimum(ty2, py2),
                    ta, pa) & pk
                supp1 = _supp_test(
                    jnp.maximum(ux1, px1), jnp.maximum(uy1, py1),
                    jnp.minimum(ux2, px2), jnp.minimum(uy2, py2),
                    ua, pa) & pk
                acc0 = acc0 & ~supp0
                acc1 = acc1 & ~supp1
            fl[pl.ds(tb, 16)] = acc0.astype(jnp.int32)
            fl[pl.ds(tb1, 16)] = acc1.astype(jnp.int32)
            return c

        lax.fori_loop(t0, t1, tgt, 0)
        return carry

    lax.fori_loop(0, nrnd, block, jnp.int32(0))

    def outw(v, c):
        b = (sid * _VPW + v) * 16
        f = fl[pl.ds(b, 16)] > 0
        ff = f.astype(jnp.float32)
        x1u[pl.ds(b, 16)] = x1s[pl.ds(b, 16)] * ff
        y1u[pl.ds(b, 16)] = y1s[pl.ds(b, 16)] * ff
        x2u[pl.ds(b, 16)] = x2s[pl.ds(b, 16)] * ff
        y2u[pl.ds(b, 16)] = y2s[pl.ds(b, 16)] * ff
        su[pl.ds(b, 16)] = jnp.where(f, ss[pl.ds(b, 16)], 0.0)
        return c

    lax.fori_loop(0, _VPW, outw, 0)
    rows = _VPW * 16
    o = sid * rows
    pltpu.sync_copy(x1u.at[pl.ds(o, rows)], ox1h.at[pl.ds(o, rows)])
    pltpu.sync_copy(y1u.at[pl.ds(o, rows)], oy1h.at[pl.ds(o, rows)])
    pltpu.sync_copy(x2u.at[pl.ds(o, rows)], ox2h.at[pl.ds(o, rows)])
    pltpu.sync_copy(y2u.at[pl.ds(o, rows)], oy2h.at[pl.ds(o, rows)])
    pltpu.sync_copy(su.at[pl.ds(o, rows)], osh.at[pl.ds(o, rows)])


@jax.jit
def kernel(boxes, scores):
    s = jnp.where(scores >= _PROB, scores, -jnp.inf)
    order = jnp.argsort(-s).astype(jnp.int32)
    pad = _NP - _N
    op = jnp.concatenate([order, jnp.arange(_N, _NP, dtype=jnp.int32)])
    sp = jnp.concatenate([s, jnp.full((pad,), -jnp.inf, jnp.float32)])
    bp = jnp.pad(boxes, ((0, pad), (0, 0)))
    x1, y1, x2, y2 = bp[:, 0], bp[:, 1], bp[:, 2], bp[:, 3]

    f32 = jax.ShapeDtypeStruct((_NP,), jnp.float32)
    call = pl.kernel(
        _nms_body,
        out_type=(f32, f32, f32, f32, f32),
        mesh=plsc.VectorSubcoreMesh(
            core_axis_name="c", subcore_axis_name="s",
            num_cores=1, num_subcores=_NS),
        scratch_types=[
            pltpu.VMEM((_NP,), jnp.float32),
            pltpu.VMEM((_NP,), jnp.float32),
            pltpu.VMEM((_NP,), jnp.float32),
            pltpu.VMEM((_NP,), jnp.float32),
            pltpu.VMEM((_NP,), jnp.float32),
            pltpu.VMEM((_NP,), jnp.int32),
            pltpu.VMEM((_NP,), jnp.float32),
            pltpu.VMEM((_NP,), jnp.float32),
            pltpu.VMEM((_NP,), jnp.float32),
            pltpu.VMEM((_NP,), jnp.float32),
            pltpu.VMEM((_NP,), jnp.float32),
            pltpu.VMEM((_NP,), jnp.float32),
            pltpu.VMEM((_NP,), jnp.int32),
            pltpu.VMEM((16,), jnp.int32),
            pltpu.VMEM((16,), jnp.int32),
            pltpu.VMEM((16,), jnp.int32),
            pltpu.SemaphoreType.DMA,
            pltpu.SemaphoreType.DMA,
            pltpu.VMEM_SHARED((_VEC, 16), jnp.int32),
            pltpu.VMEM_SHARED((_VEC, 16), jnp.int32),
        ],
        compiler_params=pltpu.CompilerParams(needs_layout_passes=False),
    )
    ox1, oy1, ox2, oy2, os_ = call(x1, y1, x2, y2, sp, op)
    kept_boxes = jnp.stack([ox1, oy1, ox2, oy2], axis=1)[:_N]
    kept_scores = os_[:_N]
    return kept_boxes, kept_scores


# speedup vs baseline: 49.6131x; 1.0012x over previous
import jax
import jax.numpy as jnp
from jax import lax
from jax.experimental import pallas as pl
from jax.experimental.pallas import tpu as pltpu
from jax.experimental.pallas import tpu_sc as plsc

_N = 5000
_NP = 5120
_VEC = _NP // 16
_NS = 16
_VPW = _VEC // _NS
_PROB = 0.5
_C24 = 2.0 ** 24
_C25 = 2.0 ** 25


def _take(v, idx):
    return v.at[idx].get(mode="promise_in_bounds")


def _supp_test(ix1, iy1, ix2, iy2, ua, ub):
    inter = jnp.maximum(ix2 - ix1, 0.0) * jnp.maximum(iy2 - iy1, 0.0)
    u = jnp.maximum(ua + ub - inter, 1e-9)
    big = inter * _C25
    a = u * _C24
    hi = a + u
    z = hi - a
    lo = u - z
    return (big > hi) | ((big == hi) & (lo < 0.0))


def _nms_body(x1h, y1h, x2h, y2h, sh, odh,
              ox1h, oy1h, ox2h, oy2h, osh,
              x1u, y1u, x2u, y2u, su, odv,
              x1s, y1s, x2s, y2s, ars, ss, fl, pubv, rda, rdb,
              sema, semb,
              sha, shb):
    sid = lax.axis_index("s")
    lanes = lax.broadcasted_iota(jnp.int32, (16,), 0)

    pltpu.sync_copy(x1h, x1u)
    pltpu.sync_copy(y1h, y1u)
    pltpu.sync_copy(x2h, x2u)
    pltpu.sync_copy(y2h, y2u)
    pltpu.sync_copy(sh, su)
    pltpu.sync_copy(odh, odv)

    def build(v, cnt):
        b = v * 16
        idx = odv[pl.ds(b, 16)]
        gx1 = plsc.load_gather(x1u, [idx])
        gy1 = plsc.load_gather(y1u, [idx])
        gx2 = plsc.load_gather(x2u, [idx])
        gy2 = plsc.load_gather(y2u, [idx])
        gs = plsc.load_gather(su, [idx])
        x1s[pl.ds(b, 16)] = gx1
        y1s[pl.ds(b, 16)] = gy1
        x2s[pl.ds(b, 16)] = gx2
        y2s[pl.ds(b, 16)] = gy2
        ss[pl.ds(b, 16)] = gs
        ars[pl.ds(b, 16)] = (gx2 - gx1) * (gy2 - gy1)
        valid = (gs >= _PROB).astype(jnp.int32)
        fl[pl.ds(b, 16)] = valid
        return cnt + jnp.sum(valid)

    nvalid = lax.fori_loop(0, _VEC, build, jnp.int32(0))
    nrnd = (nvalid + 31) // 32

    pubv[...] = jnp.zeros((16,), jnp.int32)

    def zslot(t, c):
        pltpu.sync_copy(pubv, sha.at[sid + t * _NS])
        pltpu.sync_copy(pubv, shb.at[sid + t * _NS])
        return c

    lax.fori_loop(0, _VEC // _NS, zslot, 0)
    plsc.subcore_barrier()

    def ptag(r):
        return (r + 1) + (nvalid << 9)

    def _intra16(x1v, y1v, x2v, y2v, av, f0):
        def cond(c):
            _, rem = c
            return jnp.any(rem)

        def step(c):
            f, rem = c
            iv = plsc.all_reduce_ffs(rem)
            supp = _supp_test(
                jnp.maximum(x1v, _take(x1v, iv)),
                jnp.maximum(y1v, _take(y1v, iv)),
                jnp.minimum(x2v, _take(x2v, iv)),
                jnp.minimum(y2v, _take(y2v, iv)),
                av, _take(av, iv)) & (lanes > iv)
            f = f & ~supp
            return f, f & (lanes > iv)

        ffin, _ = lax.while_loop(cond, step, (f0, f0))
        return ffin

    def block(r, carry):
        b0 = r * 32
        b1 = b0 + 16
        ax1 = x1s[pl.ds(b0, 16)]
        ay1 = y1s[pl.ds(b0, 16)]
        ax2 = x2s[pl.ds(b0, 16)]
        ay2 = y2s[pl.ds(b0, 16)]
        aa = ars[pl.ds(b0, 16)]
        cx1 = x1s[pl.ds(b1, 16)]
        cy1 = y1s[pl.ds(b1, 16)]
        cx2 = x2s[pl.ds(b1, 16)]
        cy2 = y2s[pl.ds(b1, 16)]
        ca = ars[pl.ds(b1, 16)]

        @pl.when(sid == lax.rem(r, _NS))
        def _():
            fa = _intra16(ax1, ay1, ax2, ay2, aa, fl[pl.ds(b0, 16)] > 0)
            fai = fa.astype(jnp.int32)
            accc = fl[pl.ds(b1, 16)] > 0
            for i in range(16):
                ii = jnp.full((16,), i, jnp.int32)
                supp = _supp_test(
                    jnp.maximum(cx1, _take(ax1, ii)),
                    jnp.maximum(cy1, _take(ay1, ii)),
                    jnp.minimum(cx2, _take(ax2, ii)),
                    jnp.minimum(cy2, _take(ay2, ii)),
                    ca, _take(aa, ii)) & (_take(fai, ii) > 0)
                accc = accc & ~supp
            fc = _intra16(cx1, cy1, cx2, cy2, ca, accc)
            fci = fc.astype(jnp.int32)
            fl[pl.ds(b0, 16)] = fai
            fl[pl.ds(b1, 16)] = fci
            pubv[...] = fai + (fci << 1) + (ptag(r) << 2)
            da = pltpu.async_copy(pubv, sha.at[r], sema)
            db = pltpu.async_copy(pubv, shb.at[r], semb)
            da.wait()
            db.wait()

        plsc.subcore_barrier()
        da = pltpu.async_copy(sha.at[r], rda, sema)
        db = pltpu.async_copy(shb.at[r], rdb, semb)
        da.wait()
        db.wait()
        va = rda[...]
        vb = rdb[...]
        oka = (va >> 2) == ptag(r)
        val = jnp.where(oka, va, vb)
        kfa = val & 1
        kfc = (val >> 1) & 1
        fl[pl.ds(b0, 16)] = kfa
        fl[pl.ds(b1, 16)] = kfc

        t0 = (r - sid + 16) // 16
        t1 = (nrnd - sid + 15) // 16

        def tgt(t, c):
            tb = (sid + t * 16) * 32
            tb1 = tb + 16
            tx1 = x1s[pl.ds(tb, 16)]
            ty1 = y1s[pl.ds(tb, 16)]
            tx2 = x2s[pl.ds(tb, 16)]
            ty2 = y2s[pl.ds(tb, 16)]
            ta = ars[pl.ds(tb, 16)]
            ux1 = x1s[pl.ds(tb1, 16)]
            uy1 = y1s[pl.ds(tb1, 16)]
            ux2 = x2s[pl.ds(tb1, 16)]
            uy2 = y2s[pl.ds(tb1, 16)]
            ua = ars[pl.ds(tb1, 16)]
            acc0 = fl[pl.ds(tb, 16)] > 0
            acc1 = fl[pl.ds(tb1, 16)] > 0
            for i in range(32):
                ii = jnp.full((16,), i % 16, jnp.int32)
                if i < 16:
                    sx1, sy1, sx2, sy2, sa, sf = ax1, ay1, ax2, ay2, aa, kfa
                else:
                    sx1, sy1, sx2, sy2, sa, sf = cx1, cy1, cx2, cy2, ca, kfc
                px1 = _take(sx1, ii)
                py1 = _take(sy1, ii)
                px2 = _take(sx2, ii)
                py2 = _take(sy2, ii)
                pa = _take(sa, ii)
                pk = _take(sf, ii) > 0
                supp0 = _supp_test(
                    jnp.maximum(tx1, px1), jnp.maximum(ty1, py1),
                    jnp.minimum(tx2, px2), jnp.minimum(ty2, py2),
                    ta, pa) & pk
                supp1 = _supp_test(
                    jnp.maximum(ux1, px1), jnp.maximum(uy1, py1),
                    jnp.minimum(ux2, px2), jnp.minimum(uy2, py2),
                    ua, pa) & pk
                acc0 = acc0 & ~supp0
                acc1 = acc1 & ~supp1
            fl[pl.ds(tb, 16)] = acc0.astype(jnp.int32)
            fl[pl.ds(tb1, 16)] = acc1.astype(jnp.int32)
            return c

        lax.fori_loop(t0, t1, tgt, 0)
        return carry

    lax.fori_loop(0, nrnd, block, jnp.int32(0))

    def outw(v, c):
        b = (sid * _VPW + v) * 16
        f = fl[pl.ds(b, 16)] > 0
        ff = f.astype(jnp.float32)
        x1u[pl.ds(b, 16)] = x1s[pl.ds(b, 16)] * ff
        y1u[pl.ds(b, 16)] = y1s[pl.ds(b, 16)] * ff
        x2u[pl.ds(b, 16)] = x2s[pl.ds(b, 16)] * ff
        y2u[pl.ds(b, 16)] = y2s[pl.ds(b, 16)] * ff
        su[pl.ds(b, 16)] = jnp.where(f, ss[pl.ds(b, 16)], 0.0)
        return c

    lax.fori_loop(0, _VPW, outw, 0)
    rows = _VPW * 16
    o = sid * rows
    pltpu.sync_copy(x1u.at[pl.ds(o, rows)], ox1h.at[pl.ds(o, rows)])
    pltpu.sync_copy(y1u.at[pl.ds(o, rows)], oy1h.at[pl.ds(o, rows)])
    pltpu.sync_copy(x2u.at[pl.ds(o, rows)], ox2h.at[pl.ds(o, rows)])
    pltpu.sync_copy(y2u.at[pl.ds(o, rows)], oy2h.at[pl.ds(o, rows)])
    pltpu.sync_copy(su.at[pl.ds(o, rows)], osh.at[pl.ds(o, rows)])


@jax.jit
def kernel(boxes, scores):
    s = jnp.where(scores >= _PROB, scores, -jnp.inf)
    order = jnp.argsort(-s).astype(jnp.int32)
    pad = _NP - _N
    op = jnp.concatenate([order, jnp.arange(_N, _NP, dtype=jnp.int32)])
    sp = jnp.concatenate([s, jnp.full((pad,), -jnp.inf, jnp.float32)])
    bp = jnp.pad(boxes, ((0, pad), (0, 0)))
    x1, y1, x2, y2 = bp[:, 0], bp[:, 1], bp[:, 2], bp[:, 3]

    f32 = jax.ShapeDtypeStruct((_NP,), jnp.float32)
    call = pl.kernel(
        _nms_body,
        out_type=(f32, f32, f32, f32, f32),
        mesh=plsc.VectorSubcoreMesh(
            core_axis_name="c", subcore_axis_name="s",
            num_cores=1, num_subcores=_NS),
        scratch_types=[
            pltpu.VMEM((_NP,), jnp.float32),
            pltpu.VMEM((_NP,), jnp.float32),
            pltpu.VMEM((_NP,), jnp.float32),
            pltpu.VMEM((_NP,), jnp.float32),
            pltpu.VMEM((_NP,), jnp.float32),
            pltpu.VMEM((_NP,), jnp.int32),
            pltpu.VMEM((_NP,), jnp.float32),
            pltpu.VMEM((_NP,), jnp.float32),
            pltpu.VMEM((_NP,), jnp.float32),
            pltpu.VMEM((_NP,), jnp.float32),
            pltpu.VMEM((_NP,), jnp.float32),
            pltpu.VMEM((_NP,), jnp.float32),
            pltpu.VMEM((_NP,), jnp.int32),
            pltpu.VMEM((16,), jnp.int32),
            pltpu.VMEM((16,), jnp.int32),
            pltpu.VMEM((16,), jnp.int32),
            pltpu.SemaphoreType.DMA,
            pltpu.SemaphoreType.DMA,
            pltpu.VMEM_SHARED((_VEC, 16), jnp.int32),
            pltpu.VMEM_SHARED((_VEC, 16), jnp.int32),
        ],
        compiler_params=pltpu.CompilerParams(needs_layout_passes=False),
    )
    ox1, oy1, ox2, oy2, os_ = call(x1, y1, x2, y2, sp, op)
    kept_boxes = jnp.stack([ox1, oy1, ox2, oy2], axis=1)[:_N]
    kept_scores = os_[:_N]
    return kept_boxes, kept_scores
